# baseline (speedup 1.0000x reference)
import functools

import jax
import jax.numpy as jnp
from jax import lax
from jax.experimental import pallas as pl
from jax.experimental.pallas import tpu as pltpu
from jax.experimental.pallas import tpu_sc as plsc

C = 64
NX = 512
NY = 512
NB = 4
NROWS = NB * C * NX
L = 16
NSUB = 16
NRG = NB * NX
GSUB = NRG // 32
LCAP = 4096
STRIP = 256
CAPR = 2048


def _zero_body(o_ref):
    o_ref[...] = jnp.zeros_like(o_ref)


def _make_canvas():
    return pl.pallas_call(
        _zero_body,
        out_shape=jax.ShapeDtypeStruct((NB, C, NX, NY), jnp.float32),
        grid=(NB, C // 16),
        out_specs=pl.BlockSpec((1, 16, NX, NY), lambda b, c: (b, c, 0, 0)),
    )()


def _sc_scatter_body(canvas, coords_h, feats_h, bound_h,
                     coords_v, bound_v, lbase, lp, cnt16, counts_v, pres,
                     strip_base, strip_p, cbase, cp, staging, gidx, rowtile,
                     cnt_spm, spm_base, spm_p, P, CH, G):
    cidx = lax.axis_index("c")
    sidx = lax.axis_index("s")
    iota = lax.broadcasted_iota(jnp.int32, (L,), 0)
    zeros16 = jnp.zeros((L,), jnp.int32)
    zerosf = jnp.zeros((L,), jnp.float32)
    rid = cidx * NSUB + sidx
    cview = canvas.reshape(NROWS, NY)

    pltpu.sync_copy(bound_h, bound_v)
    row0 = jnp.minimum(sidx * CH, P - CH)
    pltpu.sync_copy(coords_h.at[pl.ds(row0 * 3, CH * 3)], coords_v)
    bvec = bound_v[...]

    def scan_body(g, cnt):
        pidx = g * L + iota
        valid = (pidx < CH) & ((row0 + pidx) >= sidx * CH)
        b = plsc.load_gather(coords_v, [pidx * 3], mask=valid)
        x = plsc.load_gather(coords_v, [pidx * 3 + 1], mask=valid)
        y = plsc.load_gather(coords_v, [pidx * 3 + 2], mask=valid)
        rg = b * NX + x
        acc = valid & (b >= 0) & (b < bvec) & ((rg >> 10) == cidx)
        base = (rg << 9) | y
        pg = row0 + pidx
        plsc.store_compressed(lbase.at[pl.ds(cnt, L)], base, mask=acc)
        plsc.store_compressed(lp.at[pl.ds(cnt, L)], pg, mask=acc)
        return cnt + jnp.sum(acc.astype(jnp.int32))

    cnt = lax.fori_loop(0, G, scan_body, jnp.int32(0))

    cnt16[...] = jnp.full((L,), cnt, jnp.int32)
    pltpu.sync_copy(cnt16, cnt_spm.at[pl.ds(sidx * L, L)])

    def pub(k, _):
        pltpu.sync_copy(lbase.at[pl.ds(k * STRIP, STRIP)],
                        spm_base.at[pl.ds(sidx * LCAP + k * STRIP, STRIP)])
        pltpu.sync_copy(lp.at[pl.ds(k * STRIP, STRIP)],
                        spm_p.at[pl.ds(sidx * LCAP + k * STRIP, STRIP)])
        return 0

    lax.fori_loop(0, (cnt + STRIP - 1) // STRIP, pub, 0)
    plsc.subcore_barrier()
    pltpu.sync_copy(cnt_spm, counts_v)

    def get_count(s2):
        v = plsc.load_gather(counts_v, [s2 * L + zeros16])
        return jnp.max(v)

    def strips(per_vec, carry0):
        def s2_body(s2, carry):
            cnt2 = get_count(s2)

            def blk(k, carry):
                pltpu.sync_copy(
                    spm_base.at[pl.ds(s2 * LCAP + k * STRIP, STRIP)],
                    strip_base)
                pltpu.sync_copy(
                    spm_p.at[pl.ds(s2 * LCAP + k * STRIP, STRIP)], strip_p)
                rem = cnt2 - k * STRIP

                def vec(j, carry):
                    bse = strip_base[pl.ds(j * L, L)]
                    pp = strip_p[pl.ds(j * L, L)]
                    lv = (j * L + iota) < rem
                    return per_vec(bse, pp, lv, carry)

                return lax.fori_loop(
                    0, (jnp.minimum(rem, STRIP) + L - 1) // L, vec, carry)

            return lax.fori_loop(0, (cnt2 + STRIP - 1) // STRIP, blk, carry)

        return lax.fori_loop(0, NSUB, s2_body, carry0)

    for t in range(GSUB // L):
        pres[pl.ds(t * L, L)] = zeros16
    ones16 = zeros16 + 1

    def cache_vec(bse, pp, lv, rcnt):
        m = lv & ((bse >> 15) == rid)
        plsc.store_scatter(pres, [(bse >> 9) & (GSUB - 1)], ones16, mask=m)
        at = jnp.minimum(rcnt, CAPR)
        plsc.store_compressed(cbase.at[pl.ds(at, L)], bse, mask=m)
        plsc.store_compressed(cp.at[pl.ds(at, L)], pp, mask=m)
        return rcnt + jnp.sum(m.astype(jnp.int32))

    rcnt = strips(cache_vec, jnp.int32(0))
    in_cache = rcnt <= CAPR

    def zst(r, _):
        def zcol(t, _):
            staging[r, pl.ds(t * L, L)] = zerosf
            return 0
        lax.fori_loop(0, NY // L, zcol, 0)
        return 0

    lax.fori_loop(0, C, zst, 0)

    def member_visitor(grow, fill):
        def member_vec(bse, pp, lv, carry):
            m = lv & ((bse >> 9) == grow)

            def one(_, m):
                lane = plsc.all_reduce_ffs(m)
                yv = jnp.take_along_axis(bse & (NY - 1), lane, axis=0,
                                         mode="promise_in_bounds")
                if fill:
                    pv = jnp.take_along_axis(pp, lane, axis=0,
                                             mode="promise_in_bounds")
                    prow = jnp.max(pv)
                    pltpu.sync_copy(feats_h.at[pl.ds((prow >> 3) * 8, 8)],
                                    rowtile)
                    sub = prow & 7
                for cb in range(C // L):
                    val = (rowtile[sub, pl.ds(cb * L, L)] if fill else zerosf)
                    plsc.store_scatter(staging, [cb * L + iota, yv], val)
                return m & (iota != lane)

            nm = jnp.sum(m.astype(jnp.int32))
            lax.fori_loop(0, nm, one, m)
            return carry

        return member_vec

    def run_group_cached(g, fill):
        grow = rid * GSUB + g
        visit = member_visitor(grow, fill)

        def vec(j, _):
            bse = cbase[pl.ds(j * L, L)]
            pp = cp[pl.ds(j * L, L)]
            lv = (j * L + iota) < rcnt
            return visit(bse, pp, lv, 0)

        lax.fori_loop(0, (rcnt + L - 1) // L, vec, 0)

    def run_group_strips(g, fill):
        grow = rid * GSUB + g
        strips(member_visitor(grow, fill), 0)

    def flush_group(g):
        grow = rid * GSUB + g
        b = grow >> 9
        x = grow & (NX - 1)
        for cb in range(C // L):
            gidx[0, pl.ds(cb * L, L)] = (b * (C * NX) + x
                                         + (cb * L + iota) * NX)
        pltpu.sync_copy(staging, cview.at[gidx.at[0]])

    def group_body(g, _):
        gp = plsc.load_gather(pres, [g + zeros16])

        @pl.when((jnp.max(gp) > 0) & in_cache)
        def _():
            run_group_cached(g, True)
            flush_group(g)
            run_group_cached(g, False)

        @pl.when((jnp.max(gp) > 0) & jnp.logical_not(in_cache))
        def _():
            run_group_strips(g, True)
            flush_group(g)
            run_group_strips(g, False)

        return 0

    lax.fori_loop(0, GSUB, group_body, 0)


def _make_sc_scatter(P):
    CH = -(-((P + NSUB - 1) // NSUB) // 8) * 8
    G = (CH + L - 1) // L
    mesh = plsc.VectorSubcoreMesh(core_axis_name="c", subcore_axis_name="s")
    body = functools.partial(_sc_scatter_body, P=P, CH=CH, G=G)
    return pl.kernel(
        body,
        out_type=(),
        mesh=mesh,
        compiler_params=pltpu.CompilerParams(needs_layout_passes=False),
        scratch_types=[
            pltpu.VMEM((CH * 3,), jnp.int32),
            pltpu.VMEM((L,), jnp.int32),
            pltpu.VMEM((LCAP,), jnp.int32),
            pltpu.VMEM((LCAP,), jnp.int32),
            pltpu.VMEM((L,), jnp.int32),
            pltpu.VMEM((NSUB * L,), jnp.int32),
            pltpu.VMEM((GSUB,), jnp.int32),
            pltpu.VMEM((STRIP,), jnp.int32),
            pltpu.VMEM((STRIP,), jnp.int32),
            pltpu.VMEM((CAPR + L,), jnp.int32),
            pltpu.VMEM((CAPR + L,), jnp.int32),
            pltpu.VMEM((C, NY), jnp.float32),
            pltpu.VMEM((1, C), jnp.int32),
            pltpu.VMEM((8, C), jnp.float32),
            pltpu.VMEM_SHARED((NSUB * L,), jnp.int32),
            pltpu.VMEM_SHARED((NSUB * LCAP,), jnp.int32),
            pltpu.VMEM_SHARED((NSUB * LCAP,), jnp.int32),
        ],
    )


def kernel(voxel_features, coords, batch_size):
    P = coords.shape[0]
    bound = jnp.minimum(jnp.asarray(batch_size, jnp.int32), NB)
    bound_vec = jnp.full((L,), bound, jnp.int32)
    coords_flat = coords.astype(jnp.int32).reshape(P * 3)
    canvas = _make_canvas()
    ref = jax.new_ref(canvas)
    _make_sc_scatter(P)(ref, coords_flat, voxel_features, bound_vec)
    return ref[...]


# speedup vs baseline: 1.0852x; 1.0852x over previous
import functools

import jax
import jax.numpy as jnp
from jax import lax
from jax.experimental import pallas as pl
from jax.experimental.pallas import tpu as pltpu
from jax.experimental.pallas import tpu_sc as plsc

C = 64
NX = 512
NY = 512
NB = 4
NROWS = NB * C * NX
L = 16
NSUB = 16
NRG = NB * NX
GSUB = NRG // 32
LCAP = 4096
STRIP = 256
CAPR = 2048
PF = 32


def _zero_body(o_ref):
    o_ref[...] = jnp.zeros_like(o_ref)


def _make_canvas():
    return pl.pallas_call(
        _zero_body,
        out_shape=jax.ShapeDtypeStruct((NB, C, NX, NY), jnp.float32),
        grid=(NB, C // 8),
        out_specs=pl.BlockSpec((1, 8, NX, NY), lambda b, c: (b, c, 0, 0)),
    )()


def _sc_scatter_body(canvas, coords_h, feats_h, bound_h,
                     coords_v, bound_v, lbase, lp, cnt16, counts_v, pres,
                     strip_base, strip_p, cbase, cp, cache2b, cache2p,
                     staging, gidx, rowcache, gsem,
                     cnt_spm, spm_base, spm_p, P, CH, G):
    cidx = lax.axis_index("c")
    sidx = lax.axis_index("s")
    iota = lax.broadcasted_iota(jnp.int32, (L,), 0)
    zeros16 = jnp.zeros((L,), jnp.int32)
    zerosf = jnp.zeros((L,), jnp.float32)
    rid = cidx * NSUB + sidx
    cview = canvas.reshape(NROWS, NY)

    pltpu.sync_copy(bound_h, bound_v)
    row0 = jnp.minimum(sidx * CH, P - CH)
    pltpu.sync_copy(coords_h.at[pl.ds(row0 * 3, CH * 3)], coords_v)
    bvec = bound_v[...]

    def scan_body(g, cnt):
        pidx = g * L + iota
        valid = (pidx < CH) & ((row0 + pidx) >= sidx * CH)
        b = plsc.load_gather(coords_v, [pidx * 3], mask=valid)
        x = plsc.load_gather(coords_v, [pidx * 3 + 1], mask=valid)
        y = plsc.load_gather(coords_v, [pidx * 3 + 2], mask=valid)
        rg = b * NX + x
        acc = valid & (b >= 0) & (b < bvec) & ((rg >> 10) == cidx)
        base = (rg << 9) | y
        pg = row0 + pidx
        plsc.store_compressed(lbase.at[pl.ds(cnt, L)], base, mask=acc)
        plsc.store_compressed(lp.at[pl.ds(cnt, L)], pg, mask=acc)
        return cnt + jnp.sum(acc.astype(jnp.int32))

    cnt = lax.fori_loop(0, G, scan_body, jnp.int32(0))

    cnt16[...] = jnp.full((L,), cnt, jnp.int32)
    pltpu.sync_copy(cnt16, cnt_spm.at[pl.ds(sidx * L, L)])

    def pub(k, _):
        pltpu.sync_copy(lbase.at[pl.ds(k * STRIP, STRIP)],
                        spm_base.at[sidx, pl.ds(k * STRIP, STRIP)])
        pltpu.sync_copy(lp.at[pl.ds(k * STRIP, STRIP)],
                        spm_p.at[sidx, pl.ds(k * STRIP, STRIP)])
        return 0

    lax.fori_loop(0, (cnt + STRIP - 1) // STRIP, pub, 0)
    plsc.subcore_barrier()
    pltpu.sync_copy(cnt_spm, counts_v)

    def get_count(s2):
        v = plsc.load_gather(counts_v, [s2 * L + zeros16])
        return jnp.max(v)

    def strips(per_vec, carry0):
        def s2_body(s2, carry):
            cnt2 = get_count(s2)

            def blk(k, carry):
                pltpu.sync_copy(
                    spm_base.at[s2, pl.ds(k * STRIP, STRIP)], strip_base)
                pltpu.sync_copy(
                    spm_p.at[s2, pl.ds(k * STRIP, STRIP)], strip_p)
                rem = cnt2 - k * STRIP

                def vec(j, carry):
                    bse = strip_base[pl.ds(j * L, L)]
                    pp = strip_p[pl.ds(j * L, L)]
                    lv = (j * L + iota) < rem
                    return per_vec(bse, pp, lv, carry)

                return lax.fori_loop(
                    0, (jnp.minimum(rem, STRIP) + L - 1) // L, vec, carry)

            return lax.fori_loop(0, (cnt2 + STRIP - 1) // STRIP, blk, carry)

        return lax.fori_loop(0, NSUB, s2_body, carry0)

    for t in range(GSUB // L):
        pres[pl.ds(t * L, L)] = zeros16
    ones16 = zeros16 + 1

    def cache_vec(bse, pp, lv, rcnt):
        m = lv & ((bse >> 15) == rid)
        plsc.store_scatter(pres, [(bse >> 9) & (GSUB - 1)], ones16, mask=m)
        at = jnp.minimum(rcnt, CAPR)
        plsc.store_compressed(cbase.at[pl.ds(at, L)], bse, mask=m)
        plsc.store_compressed(cp.at[pl.ds(at, L)], pp, mask=m)
        return rcnt + jnp.sum(m.astype(jnp.int32))

    counts_all = plsc.load_gather(counts_v, [iota * L])
    fast = jnp.max(counts_all) <= STRIP

    @pl.when(fast)
    def _():
        pltpu.sync_copy(spm_base.at[:, pl.ds(0, STRIP)], cache2b)
        pltpu.sync_copy(spm_p.at[:, pl.ds(0, STRIP)], cache2p)

        def s2b(s2, rcnt):
            cnt2 = get_count(s2)

            def vec(j, rcnt):
                bse = cache2b[s2, pl.ds(j * L, L)]
                pp = cache2p[s2, pl.ds(j * L, L)]
                lv = (j * L + iota) < cnt2
                return cache_vec(bse, pp, lv, rcnt)

            return lax.fori_loop(0, (cnt2 + L - 1) // L, vec, rcnt)

        rc = lax.fori_loop(0, NSUB, s2b, jnp.int32(0))
        cnt16[...] = jnp.full((L,), rc, jnp.int32)

    @pl.when(jnp.logical_not(fast))
    def _():
        rc = strips(cache_vec, jnp.int32(0))
        cnt16[...] = jnp.full((L,), rc, jnp.int32)

    rcnt = jnp.max(cnt16[...])
    in_cache = rcnt <= CAPR

    npf = jnp.minimum(rcnt, PF)

    def pfire(i, _):
        pv = plsc.load_gather(cp, [i + zeros16])
        prow = jnp.max(pv)
        pltpu.async_copy(feats_h.at[pl.ds((prow >> 3) * 8, 8)],
                         rowcache.at[pl.ds(i * 8, 8)], gsem)
        return 0

    lax.fori_loop(0, npf, pfire, 0)

    def pdrain(i, _):
        pltpu.make_async_copy(feats_h.at[pl.ds(0, 8)],
                              rowcache.at[pl.ds(i * 8, 8)], gsem).wait()
        return 0

    lax.fori_loop(0, npf, pdrain, 0)

    def zst(r, _):
        def zcol(t, _):
            staging[r, pl.ds(t * L, L)] = zerosf
            return 0
        lax.fori_loop(0, NY // L, zcol, 0)
        return 0

    lax.fori_loop(0, C, zst, 0)

    def member_visitor(grow, fill, pos_of=None):
        def member_vec(bse, pp, lv, carry, j=None):
            m = lv & ((bse >> 9) == grow)

            def one(_, m):
                lane = plsc.all_reduce_ffs(m)
                yv = jnp.take_along_axis(bse & (NY - 1), lane, axis=0,
                                         mode="promise_in_bounds")
                if fill:
                    pv = jnp.take_along_axis(pp, lane, axis=0,
                                             mode="promise_in_bounds")
                    prow = jnp.max(pv)
                    if pos_of is not None:
                        pos = pos_of(j) + jnp.max(lane)
                        ti = jnp.minimum(pos, PF)

                        @pl.when(pos >= PF)
                        def _():
                            pltpu.sync_copy(
                                feats_h.at[pl.ds((prow >> 3) * 8, 8)],
                                rowcache.at[pl.ds(PF * 8, 8)])
                    else:
                        ti = jnp.int32(PF)
                        pltpu.sync_copy(
                            feats_h.at[pl.ds((prow >> 3) * 8, 8)],
                            rowcache.at[pl.ds(PF * 8, 8)])
                    trow = ti * 8 + (prow & 7)
                for cb in range(C // L):
                    val = (rowcache[trow, pl.ds(cb * L, L)]
                           if fill else zerosf)
                    plsc.store_scatter(staging, [cb * L + iota, yv], val)
                return m & (iota != lane)

            nm = jnp.sum(m.astype(jnp.int32))
            lax.fori_loop(0, nm, one, m)
            return carry

        return member_vec

    def run_group_cached(g, fill):
        grow = rid * GSUB + g
        visit = member_visitor(grow, fill, pos_of=lambda j: j * L)

        def vec(j, _):
            bse = cbase[pl.ds(j * L, L)]
            pp = cp[pl.ds(j * L, L)]
            lv = (j * L + iota) < rcnt
            return visit(bse, pp, lv, 0, j)

        lax.fori_loop(0, (rcnt + L - 1) // L, vec, 0)

    def run_group_strips(g, fill):
        grow = rid * GSUB + g
        strips(member_visitor(grow, fill), 0)

    def flush_group(g):
        grow = rid * GSUB + g
        b = grow >> 9
        x = grow & (NX - 1)
        for cb in range(C // L):
            gidx[0, pl.ds(cb * L, L)] = (b * (C * NX) + x
                                         + (cb * L + iota) * NX)
        pltpu.sync_copy(staging, cview.at[gidx.at[0]])

    def group_body(g, _):
        gp = plsc.load_gather(pres, [g + zeros16])

        @pl.when((jnp.max(gp) > 0) & in_cache)
        def _():
            run_group_cached(g, True)
            flush_group(g)
            run_group_cached(g, False)

        @pl.when((jnp.max(gp) > 0) & jnp.logical_not(in_cache))
        def _():
            run_group_strips(g, True)
            flush_group(g)
            run_group_strips(g, False)

        return 0

    lax.fori_loop(0, GSUB, group_body, 0)


def _make_sc_scatter(P):
    CH = -(-((P + NSUB - 1) // NSUB) // 8) * 8
    G = (CH + L - 1) // L
    mesh = plsc.VectorSubcoreMesh(core_axis_name="c", subcore_axis_name="s")
    body = functools.partial(_sc_scatter_body, P=P, CH=CH, G=G)
    return pl.kernel(
        body,
        out_type=(),
        mesh=mesh,
        compiler_params=pltpu.CompilerParams(needs_layout_passes=False),
        scratch_types=[
            pltpu.VMEM((CH * 3,), jnp.int32),
            pltpu.VMEM((L,), jnp.int32),
            pltpu.VMEM((LCAP,), jnp.int32),
            pltpu.VMEM((LCAP,), jnp.int32),
            pltpu.VMEM((L,), jnp.int32),
            pltpu.VMEM((NSUB * L,), jnp.int32),
            pltpu.VMEM((GSUB,), jnp.int32),
            pltpu.VMEM((STRIP,), jnp.int32),
            pltpu.VMEM((STRIP,), jnp.int32),
            pltpu.VMEM((CAPR + L,), jnp.int32),
            pltpu.VMEM((CAPR + L,), jnp.int32),
            pltpu.VMEM((NSUB, STRIP), jnp.int32),
            pltpu.VMEM((NSUB, STRIP), jnp.int32),
            pltpu.VMEM((C, NY), jnp.float32),
            pltpu.VMEM((1, C), jnp.int32),
            pltpu.VMEM(((PF + 1) * 8, C), jnp.float32),
            pltpu.SemaphoreType.DMA,
            pltpu.VMEM_SHARED((NSUB * L,), jnp.int32),
            pltpu.VMEM_SHARED((NSUB, LCAP), jnp.int32),
            pltpu.VMEM_SHARED((NSUB, LCAP), jnp.int32),
        ],
    )


def kernel(voxel_features, coords, batch_size):
    P = coords.shape[0]
    bound = jnp.minimum(jnp.asarray(batch_size, jnp.int32), NB)
    bound_vec = jnp.full((L,), bound, jnp.int32)
    coords_flat = coords.astype(jnp.int32).reshape(P * 3)
    canvas = _make_canvas()
    ref = jax.new_ref(canvas)
    _make_sc_scatter(P)(ref, coords_flat, voxel_features, bound_vec)
    return ref[...]


# speedup vs baseline: 1.0910x; 1.0053x over previous
import functools

import jax
import jax.numpy as jnp
from jax import lax
from jax.experimental import pallas as pl
from jax.experimental.pallas import tpu as pltpu
from jax.experimental.pallas import tpu_sc as plsc

C = 64
NX = 512
NY = 512
NB = 4
NROWS = NB * C * NX
L = 16
NSUB = 16
NRG = NB * NX
GSUB = NRG // 32
LCAP = 4096
STRIP = 256
CAPR = 2048
OCAP = CAPR + L
PF = 32


def _zero_body(o_ref):
    o_ref[...] = jnp.zeros_like(o_ref)


def _make_canvas():
    return pl.pallas_call(
        _zero_body,
        out_shape=jax.ShapeDtypeStruct((NB, C, NX, NY), jnp.float32),
        grid=(NB, C // 8),
        out_specs=pl.BlockSpec((1, 8, NX, NY), lambda b, c: (b, c, 0, 0)),
    )()


def _strips_over(flb, flp, counts_v, strip_base, strip_p, iota, cidx):
    zeros16 = jnp.zeros((L,), jnp.int32)

    def get_count(s2):
        v = plsc.load_gather(counts_v, [s2 * L + zeros16])
        return jnp.max(v)

    def strips(per_vec, carry0):
        def s2_body(s2, carry):
            cnt2 = get_count(s2)

            def blk(k, carry):
                pltpu.sync_copy(
                    flb.at[cidx * NSUB + s2, pl.ds(k * STRIP, STRIP)],
                    strip_base)
                pltpu.sync_copy(
                    flp.at[cidx * NSUB + s2, pl.ds(k * STRIP, STRIP)],
                    strip_p)
                rem = cnt2 - k * STRIP

                def vec(j, carry):
                    bse = strip_base[pl.ds(j * L, L)]
                    pp = strip_p[pl.ds(j * L, L)]
                    lv = (j * L + iota) < rem
                    return per_vec(bse, pp, lv, carry)

                return lax.fori_loop(
                    0, (jnp.minimum(rem, STRIP) + L - 1) // L, vec, carry)

            return lax.fori_loop(0, (cnt2 + STRIP - 1) // STRIP, blk, carry)

        return lax.fori_loop(0, NSUB, s2_body, carry0)

    return strips, get_count


def _sc_route_body(coords_h, bound_h, ob, op, ocnt, orc, opres, oflb, oflp,
                   coords_v, bound_v, lbase, lp, cnt16, counts_v, pres,
                   strip_base, strip_p, cbase, cp, cache2b, cache2p,
                   P, CH, G):
    cidx = lax.axis_index("c")
    sidx = lax.axis_index("s")
    iota = lax.broadcasted_iota(jnp.int32, (L,), 0)
    zeros16 = jnp.zeros((L,), jnp.int32)
    rid = cidx * NSUB + sidx

    pltpu.sync_copy(bound_h, bound_v)
    row0 = jnp.minimum(sidx * CH, P - CH)
    pltpu.sync_copy(coords_h.at[pl.ds(row0 * 3, CH * 3)], coords_v)
    bvec = bound_v[...]

    def scan_body(g, cnt):
        pidx = g * L + iota
        valid = (pidx < CH) & ((row0 + pidx) >= sidx * CH)
        b = plsc.load_gather(coords_v, [pidx * 3], mask=valid)
        x = plsc.load_gather(coords_v, [pidx * 3 + 1], mask=valid)
        y = plsc.load_gather(coords_v, [pidx * 3 + 2], mask=valid)
        rg = b * NX + x
        acc = valid & (b >= 0) & (b < bvec) & ((rg >> 10) == cidx)
        base = (rg << 9) | y
        pg = row0 + pidx
        plsc.store_compressed(lbase.at[pl.ds(cnt, L)], base, mask=acc)
        plsc.store_compressed(lp.at[pl.ds(cnt, L)], pg, mask=acc)
        return cnt + jnp.sum(acc.astype(jnp.int32))

    cnt = lax.fori_loop(0, G, scan_body, jnp.int32(0))

    cnt16[...] = jnp.full((L,), cnt, jnp.int32)
    pltpu.sync_copy(cnt16, ocnt.at[pl.ds(rid * L, L)])

    def pub(k, _):
        pltpu.sync_copy(lbase.at[pl.ds(k * STRIP, STRIP)],
                        oflb.at[rid, pl.ds(k * STRIP, STRIP)])
        pltpu.sync_copy(lp.at[pl.ds(k * STRIP, STRIP)],
                        oflp.at[rid, pl.ds(k * STRIP, STRIP)])
        return 0

    lax.fori_loop(0, (cnt + STRIP - 1) // STRIP, pub, 0)
    plsc.subcore_barrier()
    pltpu.sync_copy(ocnt.at[pl.ds(cidx * NSUB * L, NSUB * L)], counts_v)
    strips, get_count = _strips_over(oflb, oflp, counts_v,
                                     strip_base, strip_p, iota, cidx)

    for t in range(GSUB // L):
        pres[pl.ds(t * L, L)] = zeros16
    ones16 = zeros16 + 1

    def cache_vec(bse, pp, lv, rcnt):
        m = lv & ((bse >> 15) == rid)
        plsc.store_scatter(pres, [(bse >> 9) & (GSUB - 1)], ones16, mask=m)
        at = jnp.minimum(rcnt, CAPR)
        plsc.store_compressed(cbase.at[pl.ds(at, L)], bse, mask=m)
        plsc.store_compressed(cp.at[pl.ds(at, L)], pp, mask=m)
        return rcnt + jnp.sum(m.astype(jnp.int32))

    counts_all = plsc.load_gather(counts_v, [iota * L])
    fast = jnp.max(counts_all) <= STRIP

    @pl.when(fast)
    def _():
        pltpu.sync_copy(
            oflb.at[pl.ds(cidx * NSUB, NSUB), pl.ds(0, STRIP)], cache2b)
        pltpu.sync_copy(
            oflp.at[pl.ds(cidx * NSUB, NSUB), pl.ds(0, STRIP)], cache2p)

        def s2b(s2, rcnt):
            cnt2 = get_count(s2)

            def vec(j, rcnt):
                bse = cache2b[s2, pl.ds(j * L, L)]
                pp = cache2p[s2, pl.ds(j * L, L)]
                lv = (j * L + iota) < cnt2
                return cache_vec(bse, pp, lv, rcnt)

            return lax.fori_loop(0, (cnt2 + L - 1) // L, vec, rcnt)

        rc = lax.fori_loop(0, NSUB, s2b, jnp.int32(0))
        cnt16[...] = jnp.full((L,), rc, jnp.int32)

    @pl.when(jnp.logical_not(fast))
    def _():
        rc = strips(cache_vec, jnp.int32(0))
        cnt16[...] = jnp.full((L,), rc, jnp.int32)

    pltpu.sync_copy(cnt16, orc.at[pl.ds(rid * L, L)])
    pltpu.sync_copy(pres, opres.at[pl.ds(rid * GSUB, GSUB)])
    rcnt = jnp.max(cnt16[...])
    nco = jnp.minimum((rcnt + STRIP - 1) // STRIP, OCAP // STRIP)

    def cpub(k, _):
        pltpu.sync_copy(cbase.at[pl.ds(k * STRIP, STRIP)],
                        ob.at[pl.ds(rid * OCAP + k * STRIP, STRIP)])
        pltpu.sync_copy(cp.at[pl.ds(k * STRIP, STRIP)],
                        op.at[pl.ds(rid * OCAP + k * STRIP, STRIP)])
        return 0

    lax.fori_loop(0, nco, cpub, 0)


def _sc_scatter_body(canvas, feats_h, ob, op, ocnt, orc, opres, oflb, oflp,
                     cnt16, counts_v, pres, strip_base, strip_p,
                     cbase, cp, staging, gidx, rowcache, gsem, P):
    cidx = lax.axis_index("c")
    sidx = lax.axis_index("s")
    iota = lax.broadcasted_iota(jnp.int32, (L,), 0)
    zeros16 = jnp.zeros((L,), jnp.int32)
    zerosf = jnp.zeros((L,), jnp.float32)
    rid = cidx * NSUB + sidx
    cview = canvas.reshape(NROWS, NY)

    pltpu.sync_copy(orc.at[pl.ds(rid * L, L)], cnt16)
    rcnt = jnp.max(cnt16[...])
    in_cache = rcnt <= CAPR
    pltpu.sync_copy(opres.at[pl.ds(rid * GSUB, GSUB)], pres)
    pltpu.sync_copy(ocnt.at[pl.ds(cidx * NSUB * L, NSUB * L)], counts_v)
    strips, _ = _strips_over(oflb, oflp, counts_v,
                             strip_base, strip_p, iota, cidx)
    ncc = jnp.minimum((rcnt + STRIP - 1) // STRIP, OCAP // STRIP)

    def cpull(k, _):
        pltpu.sync_copy(ob.at[pl.ds(rid * OCAP + k * STRIP, STRIP)],
                        cbase.at[pl.ds(k * STRIP, STRIP)])
        pltpu.sync_copy(op.at[pl.ds(rid * OCAP + k * STRIP, STRIP)],
                        cp.at[pl.ds(k * STRIP, STRIP)])
        return 0

    lax.fori_loop(0, ncc, cpull, 0)

    npf = jnp.minimum(rcnt, PF)

    def pfire(i, _):
        pv = plsc.load_gather(cp, [i + zeros16])
        prow = jnp.max(pv)
        pltpu.async_copy(feats_h.at[pl.ds((prow >> 3) * 8, 8)],
                         rowcache.at[pl.ds(i * 8, 8)], gsem)
        return 0

    lax.fori_loop(0, npf, pfire, 0)

    def pdrain(i, _):
        pltpu.make_async_copy(feats_h.at[pl.ds(0, 8)],
                              rowcache.at[pl.ds(i * 8, 8)], gsem).wait()
        return 0

    lax.fori_loop(0, npf, pdrain, 0)

    def zst(r, _):
        def zcol(t, _):
            staging[r, pl.ds(t * L, L)] = zerosf
            return 0
        lax.fori_loop(0, NY // L, zcol, 0)
        return 0

    lax.fori_loop(0, C, zst, 0)

    def member_visitor(grow, fill, pos_of=None):
        def member_vec(bse, pp, lv, carry, j=None):
            m = lv & ((bse >> 9) == grow)

            def one(_, m):
                lane = plsc.all_reduce_ffs(m)
                yv = jnp.take_along_axis(bse & (NY - 1), lane, axis=0,
                                         mode="promise_in_bounds")
                if fill:
                    pv = jnp.take_along_axis(pp, lane, axis=0,
                                             mode="promise_in_bounds")
                    prow = jnp.max(pv)
                    if pos_of is not None:
                        pos = pos_of(j) + jnp.max(lane)
                        ti = jnp.minimum(pos, PF)

                        @pl.when(pos >= PF)
                        def _():
                            pltpu.sync_copy(
                                feats_h.at[pl.ds((prow >> 3) * 8, 8)],
                                rowcache.at[pl.ds(PF * 8, 8)])
                    else:
                        ti = jnp.int32(PF)
                        pltpu.sync_copy(
                            feats_h.at[pl.ds((prow >> 3) * 8, 8)],
                            rowcache.at[pl.ds(PF * 8, 8)])
                    trow = ti * 8 + (prow & 7)
                for cb in range(C // L):
                    val = (rowcache[trow, pl.ds(cb * L, L)]
                           if fill else zerosf)
                    plsc.store_scatter(staging, [cb * L + iota, yv], val)
                return m & (iota != lane)

            nm = jnp.sum(m.astype(jnp.int32))
            lax.fori_loop(0, nm, one, m)
            return carry

        return member_vec

    def run_group_cached(g, fill):
        grow = rid * GSUB + g
        visit = member_visitor(grow, fill, pos_of=lambda j: j * L)

        def vec(j, _):
            bse = cbase[pl.ds(j * L, L)]
            pp = cp[pl.ds(j * L, L)]
            lv = (j * L + iota) < rcnt
            return visit(bse, pp, lv, 0, j)

        lax.fori_loop(0, (rcnt + L - 1) // L, vec, 0)

    def run_group_strips(g, fill):
        grow = rid * GSUB + g
        strips(member_visitor(grow, fill), 0)

    def flush_group(g):
        grow = rid * GSUB + g
        b = grow >> 9
        x = grow & (NX - 1)
        for cb in range(C // L):
            gidx[0, pl.ds(cb * L, L)] = (b * (C * NX) + x
                                         + (cb * L + iota) * NX)
        pltpu.sync_copy(staging, cview.at[gidx.at[0]])

    def group_body(g, _):
        gp = plsc.load_gather(pres, [g + zeros16])

        @pl.when((jnp.max(gp) > 0) & in_cache)
        def _():
            run_group_cached(g, True)
            flush_group(g)
            run_group_cached(g, False)

        @pl.when((jnp.max(gp) > 0) & jnp.logical_not(in_cache))
        def _():
            run_group_strips(g, True)
            flush_group(g)
            run_group_strips(g, False)

        return 0

    lax.fori_loop(0, GSUB, group_body, 0)


def _make_sc_route(P):
    CH = -(-((P + NSUB - 1) // NSUB) // 8) * 8
    G = (CH + L - 1) // L
    mesh = plsc.VectorSubcoreMesh(core_axis_name="c", subcore_axis_name="s")
    i32 = jnp.int32
    body = functools.partial(_sc_route_body, P=P, CH=CH, G=G)
    return pl.kernel(
        body,
        out_type=(
            jax.ShapeDtypeStruct((32 * OCAP,), i32),
            jax.ShapeDtypeStruct((32 * OCAP,), i32),
            jax.ShapeDtypeStruct((32 * L,), i32),
            jax.ShapeDtypeStruct((32 * L,), i32),
            jax.ShapeDtypeStruct((32 * GSUB,), i32),
            jax.ShapeDtypeStruct((32, LCAP), i32),
            jax.ShapeDtypeStruct((32, LCAP), i32),
        ),
        mesh=mesh,
        compiler_params=pltpu.CompilerParams(needs_layout_passes=False),
        scratch_types=[
            pltpu.VMEM((CH * 3,), i32),
            pltpu.VMEM((L,), i32),
            pltpu.VMEM((LCAP,), i32),
            pltpu.VMEM((LCAP,), i32),
            pltpu.VMEM((L,), i32),
            pltpu.VMEM((NSUB * L,), i32),
            pltpu.VMEM((GSUB,), i32),
            pltpu.VMEM((STRIP,), i32),
            pltpu.VMEM((STRIP,), i32),
            pltpu.VMEM((OCAP,), i32),
            pltpu.VMEM((OCAP,), i32),
            pltpu.VMEM((NSUB, STRIP), i32),
            pltpu.VMEM((NSUB, STRIP), i32),
        ],
    )


def _make_sc_scatter(P):
    mesh = plsc.VectorSubcoreMesh(core_axis_name="c", subcore_axis_name="s")
    i32 = jnp.int32
    body = functools.partial(_sc_scatter_body, P=P)
    return pl.kernel(
        body,
        out_type=(),
        mesh=mesh,
        compiler_params=pltpu.CompilerParams(needs_layout_passes=False),
        scratch_types=[
            pltpu.VMEM((L,), i32),
            pltpu.VMEM((NSUB * L,), i32),
            pltpu.VMEM((GSUB,), i32),
            pltpu.VMEM((STRIP,), i32),
            pltpu.VMEM((STRIP,), i32),
            pltpu.VMEM((OCAP,), i32),
            pltpu.VMEM((OCAP,), i32),
            pltpu.VMEM((C, NY), jnp.float32),
            pltpu.VMEM((1, C), i32),
            pltpu.VMEM(((PF + 1) * 8, C), jnp.float32),
            pltpu.SemaphoreType.DMA,
        ],
    )


def kernel(voxel_features, coords, batch_size):
    P = coords.shape[0]
    bound = jnp.minimum(jnp.asarray(batch_size, jnp.int32), NB)
    bound_vec = jnp.full((L,), bound, jnp.int32)
    coords_flat = coords.astype(jnp.int32).reshape(P * 3)
    routed = _make_sc_route(P)(coords_flat, bound_vec)
    canvas = _make_canvas()
    ref = jax.new_ref(canvas)
    _make_sc_scatter(P)(ref, voxel_features, *routed)
    return ref[...]


# speedup vs baseline: 1.1651x; 1.0679x over previous
import functools

import jax
import jax.numpy as jnp
from jax import lax
from jax.experimental import pallas as pl
from jax.experimental.pallas import tpu as pltpu
from jax.experimental.pallas import tpu_sc as plsc

C = 64
NX = 512
NY = 512
NB = 4
NROWS = NB * C * NX
L = 16
NSUB = 16
NRG = NB * NX
GSUB = NRG // 32
LCAP = 4096
STRIP = 256
CAPR = 2048
OCAP = CAPR + L
PF = 32


def _zero_body(o_ref):
    o_ref[...] = jnp.zeros_like(o_ref)


def _make_canvas():
    return pl.pallas_call(
        _zero_body,
        out_shape=jax.ShapeDtypeStruct((NB, C, NX, NY), jnp.float32),
        grid=(NB, C // 8),
        out_specs=pl.BlockSpec((1, 8, NX, NY), lambda b, c: (b, c, 0, 0)),
    )()


def _strips_over(flb, flp, counts_v, strip_base, strip_p, iota, cidx):
    zeros16 = jnp.zeros((L,), jnp.int32)

    def get_count(s2):
        v = plsc.load_gather(counts_v, [s2 * L + zeros16])
        return jnp.max(v)

    def strips(per_vec, carry0):
        def s2_body(s2, carry):
            cnt2 = get_count(s2)

            def blk(k, carry):
                pltpu.sync_copy(
                    flb.at[cidx * NSUB + s2, pl.ds(k * STRIP, STRIP)],
                    strip_base)
                pltpu.sync_copy(
                    flp.at[cidx * NSUB + s2, pl.ds(k * STRIP, STRIP)],
                    strip_p)
                rem = cnt2 - k * STRIP

                def vec(j, carry):
                    bse = strip_base[pl.ds(j * L, L)]
                    pp = strip_p[pl.ds(j * L, L)]
                    lv = (j * L + iota) < rem
                    return per_vec(bse, pp, lv, carry)

                return lax.fori_loop(
                    0, (jnp.minimum(rem, STRIP) + L - 1) // L, vec, carry)

            return lax.fori_loop(0, (cnt2 + STRIP - 1) // STRIP, blk, carry)

        return lax.fori_loop(0, NSUB, s2_body, carry0)

    return strips, get_count


def _sc_route_body(coords_h, bound_h, ob, op, ocnt, orc, opres, oflb, oflp,
                   coords_v, bound_v, lbase, lp, cnt16, counts_v, pres,
                   strip_base, strip_p, cbase, cp, cache2b, cache2p,
                   P, CH, G):
    cidx = lax.axis_index("c")
    sidx = lax.axis_index("s")
    iota = lax.broadcasted_iota(jnp.int32, (L,), 0)
    zeros16 = jnp.zeros((L,), jnp.int32)
    rid = cidx * NSUB + sidx

    pltpu.sync_copy(bound_h, bound_v)
    row0 = jnp.minimum(sidx * CH, P - CH)
    pltpu.sync_copy(coords_h.at[pl.ds(row0 * 3, CH * 3)], coords_v)
    bvec = bound_v[...]

    def scan_body(g, cnt):
        pidx = g * L + iota
        valid = (pidx < CH) & ((row0 + pidx) >= sidx * CH)
        b = plsc.load_gather(coords_v, [pidx * 3], mask=valid)
        x = plsc.load_gather(coords_v, [pidx * 3 + 1], mask=valid)
        y = plsc.load_gather(coords_v, [pidx * 3 + 2], mask=valid)
        rg = b * NX + x
        acc = valid & (b >= 0) & (b < bvec) & ((rg >> 10) == cidx)
        base = (rg << 9) | y
        pg = row0 + pidx
        plsc.store_compressed(lbase.at[pl.ds(cnt, L)], base, mask=acc)
        plsc.store_compressed(lp.at[pl.ds(cnt, L)], pg, mask=acc)
        return cnt + jnp.sum(acc.astype(jnp.int32))

    cnt = lax.fori_loop(0, G, scan_body, jnp.int32(0))

    cnt16[...] = jnp.full((L,), cnt, jnp.int32)
    pltpu.sync_copy(cnt16, ocnt.at[pl.ds(rid * L, L)])

    def pub(k, _):
        pltpu.sync_copy(lbase.at[pl.ds(k * STRIP, STRIP)],
                        oflb.at[rid, pl.ds(k * STRIP, STRIP)])
        pltpu.sync_copy(lp.at[pl.ds(k * STRIP, STRIP)],
                        oflp.at[rid, pl.ds(k * STRIP, STRIP)])
        return 0

    lax.fori_loop(0, (cnt + STRIP - 1) // STRIP, pub, 0)
    plsc.subcore_barrier()
    pltpu.sync_copy(ocnt.at[pl.ds(cidx * NSUB * L, NSUB * L)], counts_v)
    strips, get_count = _strips_over(oflb, oflp, counts_v,
                                     strip_base, strip_p, iota, cidx)

    for t in range(GSUB // L):
        pres[pl.ds(t * L, L)] = zeros16
    ones16 = zeros16 + 1

    def cache_vec(bse, pp, lv, rcnt):
        m = lv & ((bse >> 15) == rid)
        plsc.store_scatter(pres, [(bse >> 9) & (GSUB - 1)], ones16, mask=m)
        at = jnp.minimum(rcnt, CAPR)
        plsc.store_compressed(cbase.at[pl.ds(at, L)], bse, mask=m)
        plsc.store_compressed(cp.at[pl.ds(at, L)], pp, mask=m)
        return rcnt + jnp.sum(m.astype(jnp.int32))

    counts_all = plsc.load_gather(counts_v, [iota * L])
    fast = jnp.max(counts_all) <= STRIP

    @pl.when(fast)
    def _():
        pltpu.sync_copy(
            oflb.at[pl.ds(cidx * NSUB, NSUB), pl.ds(0, STRIP)], cache2b)
        pltpu.sync_copy(
            oflp.at[pl.ds(cidx * NSUB, NSUB), pl.ds(0, STRIP)], cache2p)

        def s2b(s2, rcnt):
            cnt2 = get_count(s2)

            def vec(j, rcnt):
                bse = cache2b[s2, pl.ds(j * L, L)]
                pp = cache2p[s2, pl.ds(j * L, L)]
                lv = (j * L + iota) < cnt2
                return cache_vec(bse, pp, lv, rcnt)

            return lax.fori_loop(0, (cnt2 + L - 1) // L, vec, rcnt)

        rc = lax.fori_loop(0, NSUB, s2b, jnp.int32(0))
        cnt16[...] = jnp.full((L,), rc, jnp.int32)

    @pl.when(jnp.logical_not(fast))
    def _():
        rc = strips(cache_vec, jnp.int32(0))
        cnt16[...] = jnp.full((L,), rc, jnp.int32)

    pltpu.sync_copy(cnt16, orc.at[pl.ds(rid * L, L)])
    pltpu.sync_copy(pres, opres.at[pl.ds(rid * GSUB, GSUB)])
    rcnt = jnp.max(cnt16[...])
    nco = jnp.minimum((rcnt + STRIP - 1) // STRIP, OCAP // STRIP)

    def cpub(k, _):
        pltpu.sync_copy(cbase.at[pl.ds(k * STRIP, STRIP)],
                        ob.at[pl.ds(rid * OCAP + k * STRIP, STRIP)])
        pltpu.sync_copy(cp.at[pl.ds(k * STRIP, STRIP)],
                        op.at[pl.ds(rid * OCAP + k * STRIP, STRIP)])
        return 0

    lax.fori_loop(0, nco, cpub, 0)


def _sc_scatter_body(canvas, feats_h, ob, op, ocnt, orc, opres, oflb, oflp,
                     cnt16, counts_v, pres, strip_base, strip_p,
                     cbase, cp, staging, gidx, rowcache, gsem, P):
    cidx = lax.axis_index("c")
    sidx = lax.axis_index("s")
    iota = lax.broadcasted_iota(jnp.int32, (L,), 0)
    zeros16 = jnp.zeros((L,), jnp.int32)
    zerosf = jnp.zeros((L,), jnp.float32)
    rid = cidx * NSUB + sidx
    cview = canvas.reshape(NROWS, NY)

    pltpu.sync_copy(orc.at[pl.ds(rid * L, L)], cnt16)
    rcnt = jnp.max(cnt16[...])
    in_cache = rcnt <= CAPR
    pltpu.sync_copy(opres.at[pl.ds(rid * GSUB, GSUB)], pres)
    pltpu.sync_copy(ocnt.at[pl.ds(cidx * NSUB * L, NSUB * L)], counts_v)
    strips, _ = _strips_over(oflb, oflp, counts_v,
                             strip_base, strip_p, iota, cidx)
    ncc = jnp.minimum((rcnt + STRIP - 1) // STRIP, OCAP // STRIP)

    def cpull(k, _):
        pltpu.sync_copy(ob.at[pl.ds(rid * OCAP + k * STRIP, STRIP)],
                        cbase.at[pl.ds(k * STRIP, STRIP)])
        pltpu.sync_copy(op.at[pl.ds(rid * OCAP + k * STRIP, STRIP)],
                        cp.at[pl.ds(k * STRIP, STRIP)])
        return 0

    lax.fori_loop(0, ncc, cpull, 0)

    npf = jnp.minimum(rcnt, PF)

    def pfire(i, _):
        pv = plsc.load_gather(cp, [i + zeros16])
        prow = jnp.max(pv)
        pltpu.async_copy(feats_h.at[pl.ds((prow >> 3) * 8, 8)],
                         rowcache.at[pl.ds(i * 8, 8)], gsem)
        return 0

    lax.fori_loop(0, npf, pfire, 0)

    def pdrain(i, _):
        pltpu.make_async_copy(feats_h.at[pl.ds(0, 8)],
                              rowcache.at[pl.ds(i * 8, 8)], gsem).wait()
        return 0

    lax.fori_loop(0, npf, pdrain, 0)

    def zst(r, _):
        def zcol(t, _):
            staging[r, pl.ds(t * L, L)] = zerosf
            return 0
        lax.fori_loop(0, NY // L, zcol, 0)
        return 0

    lax.fori_loop(0, C, zst, 0)

    def member_visitor(grow, fill, pos_of=None):
        def member_vec(bse, pp, lv, qv, j=None):
            m = lv & ((bse >> 9) == grow)

            def one(_, carry):
                m, qv = carry
                lane = plsc.all_reduce_ffs(m)
                yv = jnp.take_along_axis(bse & (NY - 1), lane, axis=0,
                                         mode="promise_in_bounds")
                qv = qv | (1 << (yv >> 7))
                if fill:
                    pv = jnp.take_along_axis(pp, lane, axis=0,
                                             mode="promise_in_bounds")
                    prow = jnp.max(pv)
                    if pos_of is not None:
                        pos = pos_of(j) + jnp.max(lane)
                        ti = jnp.minimum(pos, PF)

                        @pl.when(pos >= PF)
                        def _():
                            pltpu.sync_copy(
                                feats_h.at[pl.ds((prow >> 3) * 8, 8)],
                                rowcache.at[pl.ds(PF * 8, 8)])
                    else:
                        ti = jnp.int32(PF)
                        pltpu.sync_copy(
                            feats_h.at[pl.ds((prow >> 3) * 8, 8)],
                            rowcache.at[pl.ds(PF * 8, 8)])
                    trow = ti * 8 + (prow & 7)
                for cb in range(C // L):
                    val = (rowcache[trow, pl.ds(cb * L, L)]
                           if fill else zerosf)
                    plsc.store_scatter(staging, [cb * L + iota, yv], val)
                return m & (iota != lane), qv

            nm = jnp.sum(m.astype(jnp.int32))
            _, qv = lax.fori_loop(0, nm, one, (m, qv))
            return qv

        return member_vec

    def run_group_cached(g, fill):
        grow = rid * GSUB + g
        visit = member_visitor(grow, fill, pos_of=lambda j: j * L)

        def vec(j, qv):
            bse = cbase[pl.ds(j * L, L)]
            pp = cp[pl.ds(j * L, L)]
            lv = (j * L + iota) < rcnt
            return visit(bse, pp, lv, qv, j)

        return lax.fori_loop(0, (rcnt + L - 1) // L, vec, zeros16)

    def run_group_strips(g, fill):
        grow = rid * GSUB + g
        return strips(member_visitor(grow, fill), zeros16)

    def flush_group(g, qv):
        grow = rid * GSUB + g
        b = grow >> 9
        x = grow & (NX - 1)
        for cb in range(C // L):
            gidx[0, pl.ds(cb * L, L)] = (b * (C * NX) + x
                                         + (cb * L + iota) * NX)
        qbits = jnp.max(qv)
        for q in range(NY // 128):
            @pl.when((qbits >> q) & 1 > 0)
            def _():
                pltpu.sync_copy(
                    staging.at[:, pl.ds(q * 128, 128)],
                    cview.at[gidx.at[0], pl.ds(q * 128, 128)])

    def group_body(g, _):
        gp = plsc.load_gather(pres, [g + zeros16])

        @pl.when((jnp.max(gp) > 0) & in_cache)
        def _():
            qv = run_group_cached(g, True)
            flush_group(g, qv)
            run_group_cached(g, False)

        @pl.when((jnp.max(gp) > 0) & jnp.logical_not(in_cache))
        def _():
            qv = run_group_strips(g, True)
            flush_group(g, qv)
            run_group_strips(g, False)

        return 0

    lax.fori_loop(0, GSUB, group_body, 0)


def _make_sc_route(P):
    CH = -(-((P + NSUB - 1) // NSUB) // 8) * 8
    G = (CH + L - 1) // L
    mesh = plsc.VectorSubcoreMesh(core_axis_name="c", subcore_axis_name="s")
    i32 = jnp.int32
    body = functools.partial(_sc_route_body, P=P, CH=CH, G=G)
    return pl.kernel(
        body,
        out_type=(
            jax.ShapeDtypeStruct((32 * OCAP,), i32),
            jax.ShapeDtypeStruct((32 * OCAP,), i32),
            jax.ShapeDtypeStruct((32 * L,), i32),
            jax.ShapeDtypeStruct((32 * L,), i32),
            jax.ShapeDtypeStruct((32 * GSUB,), i32),
            jax.ShapeDtypeStruct((32, LCAP), i32),
            jax.ShapeDtypeStruct((32, LCAP), i32),
        ),
        mesh=mesh,
        compiler_params=pltpu.CompilerParams(needs_layout_passes=False),
        scratch_types=[
            pltpu.VMEM((CH * 3,), i32),
            pltpu.VMEM((L,), i32),
            pltpu.VMEM((LCAP,), i32),
            pltpu.VMEM((LCAP,), i32),
            pltpu.VMEM((L,), i32),
            pltpu.VMEM((NSUB * L,), i32),
            pltpu.VMEM((GSUB,), i32),
            pltpu.VMEM((STRIP,), i32),
            pltpu.VMEM((STRIP,), i32),
            pltpu.VMEM((OCAP,), i32),
            pltpu.VMEM((OCAP,), i32),
            pltpu.VMEM((NSUB, STRIP), i32),
            pltpu.VMEM((NSUB, STRIP), i32),
        ],
    )


def _make_sc_scatter(P):
    mesh = plsc.VectorSubcoreMesh(core_axis_name="c", subcore_axis_name="s")
    i32 = jnp.int32
    body = functools.partial(_sc_scatter_body, P=P)
    return pl.kernel(
        body,
        out_type=(),
        mesh=mesh,
        compiler_params=pltpu.CompilerParams(needs_layout_passes=False),
        scratch_types=[
            pltpu.VMEM((L,), i32),
            pltpu.VMEM((NSUB * L,), i32),
            pltpu.VMEM((GSUB,), i32),
            pltpu.VMEM((STRIP,), i32),
            pltpu.VMEM((STRIP,), i32),
            pltpu.VMEM((OCAP,), i32),
            pltpu.VMEM((OCAP,), i32),
            pltpu.VMEM((C, NY), jnp.float32),
            pltpu.VMEM((1, C), i32),
            pltpu.VMEM(((PF + 1) * 8, C), jnp.float32),
            pltpu.SemaphoreType.DMA,
        ],
    )


def kernel(voxel_features, coords, batch_size):
    P = coords.shape[0]
    bound = jnp.minimum(jnp.asarray(batch_size, jnp.int32), NB)
    bound_vec = jnp.full((L,), bound, jnp.int32)
    coords_flat = coords.astype(jnp.int32).reshape(P * 3)
    routed = _make_sc_route(P)(coords_flat, bound_vec)
    canvas = _make_canvas()
    ref = jax.new_ref(canvas)
    _make_sc_scatter(P)(ref, voxel_features, *routed)
    return ref[...]
